# baseline (speedup 1.0000x reference)
import functools

import jax
import jax.numpy as jnp
from jax import lax
from jax.experimental import pallas as pl
from jax.experimental.pallas import tpu as pltpu
from jax.experimental.pallas import tpu_sc as plsc

N = 10000
E = 320000
D_IN = 128
H = 64
K = 8

NC = 2
NS = 16
HH = H // 2
CH = 128
NCH = 80
NCH2 = 160
WCH = 40
EP = NC * NS * NCH * CH
EP2 = NS * NCH2 * CH
TRASH = N
ACC = 10112
ZROWS = ACC // NS
WROWS = 624
RC = 104
TAIL = N - NS * WROWS


@functools.cache
def _mesh():
    return plsc.VectorSubcoreMesh(core_axis_name="c", subcore_axis_name="s",
                                  num_cores=NC, num_subcores=NS)


_SC_PARAMS = pltpu.CompilerParams(use_tc_tiling_on_sc=False)


def _sc_deg(src4, ones_16, zeros_16):
    return pl.kernel(
        _sc_deg_body,
        mesh=_mesh(),
        out_type=jax.ShapeDtypeStruct((NC, N, 16), jnp.float32),
        scratch_types=[
            pltpu.VMEM((NCH, CH), jnp.int32),
            pltpu.VMEM((CH, 16), jnp.float32),
            pltpu.VMEM_SHARED((ACC, 16), jnp.float32),
        ],
        compiler_params=_SC_PARAMS,
    )(src4, ones_16, zeros_16)


def _sc_deg_body(src_hbm, ones_hbm, zeros_hbm, d_out, isrc, ones_v, acc):
    c = lax.axis_index("c")
    s = lax.axis_index("s")
    pltpu.sync_copy(zeros_hbm.at[pl.ds(s * ZROWS, ZROWS)],
                    acc.at[pl.ds(s * ZROWS, ZROWS)])
    pltpu.sync_copy(src_hbm.at[c, s], isrc)
    pltpu.sync_copy(ones_hbm, ones_v)
    plsc.subcore_barrier()

    @pl.loop(0, NCH)
    def _(j):
        pltpu.sync_copy(ones_v, acc.at[isrc.at[j]], add=True)

    plsc.subcore_barrier()
    pltpu.sync_copy(acc.at[pl.ds(s * WROWS, WROWS)],
                    d_out.at[c, pl.ds(s * WROWS, WROWS)])

    @pl.when(s == 0)
    def _():
        pltpu.sync_copy(acc.at[pl.ds(NS * WROWS, TAIL)],
                        d_out.at[c, pl.ds(NS * WROWS, TAIL)])


def _sc_mega(a2, dinv, src3, dst3, bias2):
    return pl.kernel(
        _sc_mega_body,
        mesh=_mesh(),
        out_type=jax.ShapeDtypeStruct((NC, N, HH), jnp.float32),
        scratch_types=[
            pltpu.VMEM((WCH, CH), jnp.int32),
            pltpu.VMEM((WCH, CH), jnp.int32),
            pltpu.VMEM((CH, HH), jnp.float32),
            pltpu.VMEM((CH, HH), jnp.float32),
            pltpu.VMEM((RC, HH), jnp.float32),
            pltpu.VMEM((RC, HH), jnp.float32),
            pltpu.VMEM((RC, HH), jnp.float32),
            pltpu.VMEM((RC, HH), jnp.float32),
            pltpu.VMEM((RC, 16), jnp.float32),
            pltpu.VMEM((1, HH), jnp.float32),
            pltpu.VMEM((WROWS + TAIL, HH), jnp.float32),
            pltpu.VMEM((WROWS + TAIL, HH), jnp.float32),
            pltpu.VMEM_SHARED((ACC, HH), jnp.float32),
            pltpu.VMEM_SHARED((N, HH), jnp.float32),
            pltpu.SemaphoreType.DMA,
            pltpu.SemaphoreType.DMA,
            pltpu.SemaphoreType.DMA,
            pltpu.SemaphoreType.DMA,
            pltpu.SemaphoreType.DMA,
            pltpu.SemaphoreType.DMA,
        ],
        compiler_params=_SC_PARAMS,
    )(a2, dinv, src3, dst3, bias2)


def _sc_mega_body(a_hbm, dinv_hbm, src_hbm, dst_hbm, bias_hbm,
                  out_hbm, isrc, idst, gb0, gb1,
                  s_v, a_v, g_v, z_v, d_v, bias_v, bA, bB,
                  acc, gsh,
                  gs0, gs1, gs2, gs3, ss0, ss1):
    c = lax.axis_index("c")
    s = lax.axis_index("s")
    gbufs = (gb0, gb1)
    gsems = (gs0, gs1)
    ssems = (ss0, ss1)

    def row_chunks(body):
        for m in range(WROWS // RC):
            body(s * WROWS + m * RC, m * RC, RC)

        @pl.when(s == 0)
        def _():
            body(NS * WROWS, WROWS, TAIL)

    def combine(k, bslot, first, final):
        def one_chunk(r0, l0, rows):
            ca = pltpu.async_copy(a_hbm.at[k, c, pl.ds(r0, rows)],
                                  a_v.at[pl.ds(0, rows)], gs0)
            cd = pltpu.async_copy(dinv_hbm.at[pl.ds(r0, rows)],
                                  d_v.at[pl.ds(0, rows)], gs1)
            if not first:
                cs = pltpu.async_copy(acc.at[pl.ds(r0, rows)],
                                      s_v.at[pl.ds(0, rows)], gs2)
            ca.wait()
            cd.wait()
            if not first:
                cs.wait()
                cz = pltpu.async_copy(z_v.at[pl.ds(0, rows)],
                                      acc.at[pl.ds(r0, rows)], gs2)

            @pl.loop(0, rows)
            def _(r):
                d = d_v[r, pl.ds(0, 16)]
                for q in range(2):
                    sl = pl.ds(q * 16, 16)
                    if final:
                        o = a_v[r, sl] - d * s_v[r, sl] \
                            - bslot[l0 + r, sl] + bias_v[0, sl]
                        g_v[r, sl] = jnp.maximum(o, 0.0)
                    else:
                        if first:
                            bkv = a_v[r, sl] - bslot[l0 + r, sl]
                        else:
                            bkv = a_v[r, sl] - (2.0 * d) * s_v[r, sl] \
                                - bslot[l0 + r, sl]
                        bslot[l0 + r, sl] = bkv
                        g_v[r, sl] = d * bkv

            if final:
                pltpu.sync_copy(g_v.at[pl.ds(0, rows)],
                                out_hbm.at[c, pl.ds(r0, rows)])
            else:
                pltpu.sync_copy(g_v.at[pl.ds(0, rows)],
                                gsh.at[pl.ds(r0, rows)])
            if not first:
                cz.wait()

        row_chunks(one_chunk)

    def scatter_phase():
        @pl.loop(0, NCH2 // WCH)
        def _(w):
            cw0 = pltpu.async_copy(src_hbm.at[s, pl.ds(w * WCH, WCH)],
                                   isrc, gs2)
            cw1 = pltpu.async_copy(dst_hbm.at[s, pl.ds(w * WCH, WCH)],
                                   idst, gs3)
            cw0.wait()
            cw1.wait()

            @pl.loop(0, WCH // 2)
            def _(jj):
                j0 = jj * 2
                for i in range(2):
                    @pl.when(jj > 0)
                    def _(i=i):
                        pltpu.make_async_copy(gbufs[i],
                                              acc.at[idst.at[j0 - 2 + i]],
                                              ssems[i]).wait()
                    pltpu.async_copy(gsh.at[isrc.at[j0 + i]], gbufs[i],
                                     gsems[i])
                for i in range(2):
                    pltpu.make_async_copy(gsh.at[isrc.at[j0 + i]], gbufs[i],
                                          gsems[i]).wait()
                    pltpu.async_copy(gbufs[i], acc.at[idst.at[j0 + i]],
                                     ssems[i], add=True)

            for i in range(2):
                pltpu.make_async_copy(gbufs[i], acc.at[idst.at[WCH - 2 + i]],
                                      ssems[i]).wait()

    pltpu.sync_copy(bias_hbm.at[pl.ds(c, 1)], bias_v)

    @pl.loop(0, RC)
    def _(r):
        z16 = jnp.zeros((16,), jnp.float32)
        for q in range(2):
            z_v[r, pl.ds(q * 16, 16)] = z16

    @pl.loop(0, WROWS + TAIL)
    def _(r):
        z16 = jnp.zeros((16,), jnp.float32)
        for q in range(2):
            bA[r, pl.ds(q * 16, 16)] = z16
            bB[r, pl.ds(q * 16, 16)] = z16

    def zero_chunk(r0, l0, rows):
        pltpu.sync_copy(z_v.at[pl.ds(0, rows)], acc.at[pl.ds(r0, rows)])

    row_chunks(zero_chunk)
    combine(K - 1, bA, first=True, final=False)
    plsc.subcore_barrier()

    @pl.loop(0, (K - 2) // 2)
    def _(jj):
        k_even = K - 2 - 2 * jj
        scatter_phase()
        plsc.subcore_barrier()
        combine(k_even, bB, first=False, final=False)
        plsc.subcore_barrier()
        scatter_phase()
        plsc.subcore_barrier()
        combine(k_even - 1, bA, first=False, final=False)
        plsc.subcore_barrier()

    scatter_phase()
    plsc.subcore_barrier()
    combine(0, bB, first=False, final=True)


BM = 2000
BD = 2000


def _mm_body(x_ref, w_ref, o_ref):
    o_ref[0, 0] = jnp.dot(x_ref[...], w_ref[0, 0],
                          preferred_element_type=jnp.float32)


def _matmul(x, w2):
    return pl.pallas_call(
        _mm_body,
        grid=(N // BM, K, NC),
        in_specs=[
            pl.BlockSpec((BM, D_IN), lambda i, k, c: (i, 0)),
            pl.BlockSpec((1, 1, D_IN, HH), lambda i, k, c: (k, c, 0, 0)),
        ],
        out_specs=pl.BlockSpec((1, 1, BM, HH), lambda i, k, c: (k, c, i, 0)),
        out_shape=jax.ShapeDtypeStruct((K, NC, N, HH), jnp.float32),
    )(x, w2)


def _pre_body(deg_ref, dinv_ref):
    deg = deg_ref[0, :, 0:1] + deg_ref[1, :, 0:1]
    dinv = jnp.where(deg > 0, lax.rsqrt(jnp.maximum(deg, 1.0)), 0.0)
    dinv_ref[...] = jnp.broadcast_to(dinv, dinv_ref.shape)


def _pre(deg_parts):
    return pl.pallas_call(
        _pre_body,
        grid=(N // BD,),
        in_specs=[pl.BlockSpec((NC, BD, 16), lambda i: (0, i, 0))],
        out_specs=pl.BlockSpec((BD, 16), lambda i: (i, 0)),
        out_shape=jax.ShapeDtypeStruct((N, 16), jnp.float32),
    )(deg_parts)


def kernel(x, edge_index, W, b):
    src = edge_index[0].astype(jnp.int32)
    dst = edge_index[1].astype(jnp.int32)
    pad = EP - E
    src_s = jnp.concatenate([src, jnp.full((pad,), TRASH, jnp.int32)]
                            ).reshape(NC, NS, NCH, CH)
    pad2 = EP2 - E
    src_g = jnp.concatenate([src, jnp.zeros((pad2,), jnp.int32)]
                            ).reshape(NS, NCH2, CH)
    dst_s = jnp.concatenate([dst, jnp.full((pad2,), TRASH, jnp.int32)]
                            ).reshape(NS, NCH2, CH)

    w2 = W.reshape(K, D_IN, NC, HH).transpose(0, 2, 1, 3)
    bias2 = b.reshape(NC, HH)
    zeros_16 = jnp.zeros((ACC, 16), jnp.float32)
    ones_16 = jnp.ones((CH, 16), jnp.float32)

    deg_parts = _sc_deg(src_s, ones_16, zeros_16)
    a2 = _matmul(x, w2)
    dinv = _pre(deg_parts)
    halves = _sc_mega(a2, dinv, src_g, dst_s, bias2)
    return jnp.concatenate([halves[0], halves[1]], axis=1)


# speedup vs baseline: 1.0208x; 1.0208x over previous
import functools

import jax
import jax.numpy as jnp
from jax import lax
from jax.experimental import pallas as pl
from jax.experimental.pallas import tpu as pltpu
from jax.experimental.pallas import tpu_sc as plsc

N = 10000
E = 320000
D_IN = 128
H = 64
K = 8

NC = 2
NS = 16
HH = H // 2
CH = 128
NCH = 80
NCH2 = 160
WCH = 40
EP = NC * NS * NCH * CH
EP2 = NS * NCH2 * CH
TRASH = N
ACC = 10112
ZROWS = ACC // NS
WROWS = 624
RC = 104
TAIL = N - NS * WROWS


@functools.cache
def _mesh():
    return plsc.VectorSubcoreMesh(core_axis_name="c", subcore_axis_name="s",
                                  num_cores=NC, num_subcores=NS)


_SC_PARAMS = pltpu.CompilerParams(use_tc_tiling_on_sc=False)


def _sc_deg(src4, ones_16, zeros_16):
    return pl.kernel(
        _sc_deg_body,
        mesh=_mesh(),
        out_type=jax.ShapeDtypeStruct((NC, N, 16), jnp.float32),
        scratch_types=[
            pltpu.VMEM((NCH, CH), jnp.int32),
            pltpu.VMEM((CH, 16), jnp.float32),
            pltpu.VMEM_SHARED((ACC, 16), jnp.float32),
        ],
        compiler_params=_SC_PARAMS,
    )(src4, ones_16, zeros_16)


def _sc_deg_body(src_hbm, ones_hbm, zeros_hbm, d_out, isrc, ones_v, acc):
    c = lax.axis_index("c")
    s = lax.axis_index("s")
    pltpu.sync_copy(zeros_hbm.at[pl.ds(s * ZROWS, ZROWS)],
                    acc.at[pl.ds(s * ZROWS, ZROWS)])
    pltpu.sync_copy(src_hbm.at[c, s], isrc)
    pltpu.sync_copy(ones_hbm, ones_v)
    plsc.subcore_barrier()

    @pl.loop(0, NCH)
    def _(j):
        pltpu.sync_copy(ones_v, acc.at[isrc.at[j]], add=True)

    plsc.subcore_barrier()
    pltpu.sync_copy(acc.at[pl.ds(s * WROWS, WROWS)],
                    d_out.at[c, pl.ds(s * WROWS, WROWS)])

    @pl.when(s == 0)
    def _():
        pltpu.sync_copy(acc.at[pl.ds(NS * WROWS, TAIL)],
                        d_out.at[c, pl.ds(NS * WROWS, TAIL)])


def _sc_mega(a2, dinv, src3, dst3, bias2):
    return pl.kernel(
        _sc_mega_body,
        mesh=_mesh(),
        out_type=jax.ShapeDtypeStruct((NC, N, HH), jnp.float32),
        scratch_types=[
            pltpu.VMEM((WCH, CH), jnp.int32),
            pltpu.VMEM((WCH, CH), jnp.int32),
            pltpu.VMEM((WCH, CH), jnp.int32),
            pltpu.VMEM((WCH, CH), jnp.int32),
            pltpu.VMEM((CH, HH), jnp.float32),
            pltpu.VMEM((CH, HH), jnp.float32),
            pltpu.VMEM((RC, HH), jnp.float32),
            pltpu.VMEM((RC, HH), jnp.float32),
            pltpu.VMEM((RC, HH), jnp.float32),
            pltpu.VMEM((RC, HH), jnp.float32),
            pltpu.VMEM((RC, 16), jnp.float32),
            pltpu.VMEM((1, HH), jnp.float32),
            pltpu.VMEM((WROWS + TAIL, HH), jnp.float32),
            pltpu.VMEM((WROWS + TAIL, HH), jnp.float32),
            pltpu.VMEM_SHARED((ACC, HH), jnp.float32),
            pltpu.VMEM_SHARED((N, HH), jnp.float32),
            pltpu.SemaphoreType.DMA,
            pltpu.SemaphoreType.DMA,
            pltpu.SemaphoreType.DMA,
            pltpu.SemaphoreType.DMA,
            pltpu.SemaphoreType.DMA,
            pltpu.SemaphoreType.DMA,
        ],
        compiler_params=_SC_PARAMS,
    )(a2, dinv, src3, dst3, bias2)


def _sc_mega_body(a_hbm, dinv_hbm, src_hbm, dst_hbm, bias_hbm,
                  out_hbm, isrc, idst, isrc1, idst1, gb0, gb1,
                  s_v, a_v, g_v, z_v, d_v, bias_v, bA, bB,
                  acc, gsh,
                  gs0, gs1, gs2, gs3, ss0, ss1):
    c = lax.axis_index("c")
    s = lax.axis_index("s")
    gbufs = (gb0, gb1)
    gsems = (gs0, gs1)
    ssems = (ss0, ss1)

    def row_chunks(body):
        for m in range(WROWS // RC):
            body(s * WROWS + m * RC, m * RC, RC)

        @pl.when(s == 0)
        def _():
            body(NS * WROWS, WROWS, TAIL)

    def combine(k, bslot, first, final):
        def one_chunk(r0, l0, rows):
            ca = pltpu.async_copy(a_hbm.at[k, c, pl.ds(r0, rows)],
                                  a_v.at[pl.ds(0, rows)], gs0)
            cd = pltpu.async_copy(dinv_hbm.at[pl.ds(r0, rows)],
                                  d_v.at[pl.ds(0, rows)], gs1)
            if not first:
                cs = pltpu.async_copy(acc.at[pl.ds(r0, rows)],
                                      s_v.at[pl.ds(0, rows)], gs2)
            ca.wait()
            cd.wait()
            if not first:
                cs.wait()
                cz = pltpu.async_copy(z_v.at[pl.ds(0, rows)],
                                      acc.at[pl.ds(r0, rows)], gs2)

            @pl.loop(0, rows)
            def _(r):
                d = d_v[r, pl.ds(0, 16)]
                for q in range(2):
                    sl = pl.ds(q * 16, 16)
                    if final:
                        o = a_v[r, sl] - d * s_v[r, sl] \
                            - bslot[l0 + r, sl] + bias_v[0, sl]
                        g_v[r, sl] = jnp.maximum(o, 0.0)
                    else:
                        if first:
                            bkv = a_v[r, sl] - bslot[l0 + r, sl]
                        else:
                            bkv = a_v[r, sl] - (2.0 * d) * s_v[r, sl] \
                                - bslot[l0 + r, sl]
                        bslot[l0 + r, sl] = bkv
                        g_v[r, sl] = d * bkv

            if final:
                pltpu.sync_copy(g_v.at[pl.ds(0, rows)],
                                out_hbm.at[c, pl.ds(r0, rows)])
            else:
                pltpu.sync_copy(g_v.at[pl.ds(0, rows)],
                                gsh.at[pl.ds(r0, rows)])
            if not first:
                cz.wait()

        row_chunks(one_chunk)

    def scatter_phase():
        iwin = ((isrc, idst), (isrc1, idst1))

        def window(ws, wd):
            @pl.loop(0, WCH // 2)
            def _(jj):
                j0 = jj * 2
                for i in range(2):
                    @pl.when(jj > 0)
                    def _(i=i):
                        pltpu.make_async_copy(gbufs[i],
                                              acc.at[wd.at[j0 - 2 + i]],
                                              ssems[i]).wait()
                    pltpu.async_copy(gsh.at[ws.at[j0 + i]], gbufs[i],
                                     gsems[i])
                for i in range(2):
                    pltpu.make_async_copy(gsh.at[ws.at[j0 + i]], gbufs[i],
                                          gsems[i]).wait()
                    pltpu.async_copy(gbufs[i], acc.at[wd.at[j0 + i]],
                                     ssems[i], add=True)

            for i in range(2):
                pltpu.make_async_copy(gbufs[i], acc.at[wd.at[WCH - 2 + i]],
                                      ssems[i]).wait()

        def load(w, ws, wd):
            return (pltpu.async_copy(src_hbm.at[s, pl.ds(w * WCH, WCH)],
                                     ws, gs2),
                    pltpu.async_copy(dst_hbm.at[s, pl.ds(w * WCH, WCH)],
                                     wd, gs3))

        nwin = NCH2 // WCH
        for cp in load(0, *iwin[0]):
            cp.wait()
        for w in range(nwin):
            if w + 1 < nwin:
                nxt = load(w + 1, *iwin[(w + 1) % 2])
            window(*iwin[w % 2])
            if w + 1 < nwin:
                for cp in nxt:
                    cp.wait()

    pltpu.sync_copy(bias_hbm.at[pl.ds(c, 1)], bias_v)

    @pl.loop(0, RC)
    def _(r):
        z16 = jnp.zeros((16,), jnp.float32)
        for q in range(2):
            z_v[r, pl.ds(q * 16, 16)] = z16

    @pl.loop(0, WROWS + TAIL)
    def _(r):
        z16 = jnp.zeros((16,), jnp.float32)
        for q in range(2):
            bA[r, pl.ds(q * 16, 16)] = z16
            bB[r, pl.ds(q * 16, 16)] = z16

    def zero_chunk(r0, l0, rows):
        pltpu.sync_copy(z_v.at[pl.ds(0, rows)], acc.at[pl.ds(r0, rows)])

    row_chunks(zero_chunk)
    combine(K - 1, bA, first=True, final=False)
    plsc.subcore_barrier()

    @pl.loop(0, (K - 2) // 2)
    def _(jj):
        k_even = K - 2 - 2 * jj
        scatter_phase()
        plsc.subcore_barrier()
        combine(k_even, bB, first=False, final=False)
        plsc.subcore_barrier()
        scatter_phase()
        plsc.subcore_barrier()
        combine(k_even - 1, bA, first=False, final=False)
        plsc.subcore_barrier()

    scatter_phase()
    plsc.subcore_barrier()
    combine(0, bB, first=False, final=True)


BM = 2000
BD = 2000


def _mm_body(x_ref, w_ref, o_ref):
    o_ref[0, 0] = jnp.dot(x_ref[...], w_ref[0, 0],
                          preferred_element_type=jnp.float32)


def _matmul(x, w2):
    return pl.pallas_call(
        _mm_body,
        grid=(N // BM, K, NC),
        in_specs=[
            pl.BlockSpec((BM, D_IN), lambda i, k, c: (i, 0)),
            pl.BlockSpec((1, 1, D_IN, HH), lambda i, k, c: (k, c, 0, 0)),
        ],
        out_specs=pl.BlockSpec((1, 1, BM, HH), lambda i, k, c: (k, c, i, 0)),
        out_shape=jax.ShapeDtypeStruct((K, NC, N, HH), jnp.float32),
    )(x, w2)


def _pre_body(deg_ref, dinv_ref):
    deg = deg_ref[0, :, 0:1] + deg_ref[1, :, 0:1]
    dinv = jnp.where(deg > 0, lax.rsqrt(jnp.maximum(deg, 1.0)), 0.0)
    dinv_ref[...] = jnp.broadcast_to(dinv, dinv_ref.shape)


def _pre(deg_parts):
    return pl.pallas_call(
        _pre_body,
        grid=(N // BD,),
        in_specs=[pl.BlockSpec((NC, BD, 16), lambda i: (0, i, 0))],
        out_specs=pl.BlockSpec((BD, 16), lambda i: (i, 0)),
        out_shape=jax.ShapeDtypeStruct((N, 16), jnp.float32),
    )(deg_parts)


def kernel(x, edge_index, W, b):
    src = edge_index[0].astype(jnp.int32)
    dst = edge_index[1].astype(jnp.int32)
    pad = EP - E
    src_s = jnp.concatenate([src, jnp.full((pad,), TRASH, jnp.int32)]
                            ).reshape(NC, NS, NCH, CH)
    pad2 = EP2 - E
    src_g = jnp.concatenate([src, jnp.zeros((pad2,), jnp.int32)]
                            ).reshape(NS, NCH2, CH)
    dst_s = jnp.concatenate([dst, jnp.full((pad2,), TRASH, jnp.int32)]
                            ).reshape(NS, NCH2, CH)

    w2 = W.reshape(K, D_IN, NC, HH).transpose(0, 2, 1, 3)
    bias2 = b.reshape(NC, HH)
    zeros_16 = jnp.zeros((ACC, 16), jnp.float32)
    ones_16 = jnp.ones((CH, 16), jnp.float32)

    deg_parts = _sc_deg(src_s, ones_16, zeros_16)
    a2 = _matmul(x, w2)
    dinv = _pre(deg_parts)
    halves = _sc_mega(a2, dinv, src_g, dst_s, bias2)
    return jnp.concatenate([halves[0], halves[1]], axis=1)


# speedup vs baseline: 1.0478x; 1.0265x over previous
import functools

import jax
import jax.numpy as jnp
from jax import lax
from jax.experimental import pallas as pl
from jax.experimental.pallas import tpu as pltpu
from jax.experimental.pallas import tpu_sc as plsc

N = 10000
E = 320000
D_IN = 128
H = 64
K = 8

NC = 2
NS = 16
CH = 128
NCH = 80
NCH2 = 158
HH = H // 2
EP2 = NS * NCH2 * CH
EP = NC * NS * NCH * CH
TRASH = N
ACC = 10112
ZROWS = ACC // NS
WROWS = 624

@functools.cache
def _mesh():
    return plsc.VectorSubcoreMesh(core_axis_name="c", subcore_axis_name="s",
                                  num_cores=NC, num_subcores=NS)


_SC_PARAMS = pltpu.CompilerParams(use_tc_tiling_on_sc=False)


def _sc_prop(g, src4, dst4, zeros_h):
    return pl.kernel(
        _sc_prop_body,
        mesh=_mesh(),
        out_type=jax.ShapeDtypeStruct((NC, N, HH), jnp.float32),
        scratch_types=[
            pltpu.VMEM((NCH2, CH), jnp.int32),
            pltpu.VMEM((NCH2, CH), jnp.int32),
            pltpu.VMEM((CH, HH), jnp.float32),
            pltpu.VMEM((CH, HH), jnp.float32),
            pltpu.VMEM((CH, HH), jnp.float32),
            pltpu.VMEM((CH, HH), jnp.float32),
            pltpu.VMEM_SHARED((ACC, HH), jnp.float32),
            pltpu.VMEM_SHARED((N, HH), jnp.float32),
            pltpu.SemaphoreType.DMA,
            pltpu.SemaphoreType.DMA,
            pltpu.SemaphoreType.DMA,
            pltpu.SemaphoreType.DMA,
            pltpu.SemaphoreType.DMA,
            pltpu.SemaphoreType.DMA,
            pltpu.SemaphoreType.DMA,
            pltpu.SemaphoreType.DMA,
        ],
        compiler_params=_SC_PARAMS,
    )(g, src4, dst4, zeros_h)


def _sc_prop_body(g_hbm, src_hbm, dst_hbm, zeros_hbm, s_out,
                  isrc, idst, gb0, gb1, gb2, gb3, acc, gsh,
                  gs0, gs1, gs2, gs3, ss0, ss1, ss2, ss3):
    gbufs = (gb0, gb1, gb2, gb3)
    gsems = (gs0, gs1, gs2, gs3)
    ssems = (ss0, ss1, ss2, ss3)
    c = lax.axis_index("c")
    s = lax.axis_index("s")
    cz = pltpu.async_copy(zeros_hbm.at[pl.ds(s * ZROWS, ZROWS)],
                          acc.at[pl.ds(s * ZROWS, ZROWS)], gs0)
    cg = pltpu.async_copy(g_hbm.at[c, pl.ds(s * WROWS, WROWS)],
                          gsh.at[pl.ds(s * WROWS, WROWS)], gs1)
    ci = pltpu.async_copy(src_hbm.at[s], isrc, gs2)
    cj = pltpu.async_copy(dst_hbm.at[s], idst, gs3)

    @pl.when(s == 0)
    def _():
        pltpu.sync_copy(g_hbm.at[c, pl.ds(NS * WROWS, N - NS * WROWS)],
                        gsh.at[pl.ds(NS * WROWS, N - NS * WROWS)])

    cz.wait()
    cg.wait()
    ci.wait()
    cj.wait()
    plsc.subcore_barrier()

    @pl.loop(0, NCH2 // 2)
    def _(jj):
        j0 = jj * 2
        for i in range(2):
            @pl.when(jj > 0)
            def _(i=i):
                pltpu.make_async_copy(gbufs[i], acc.at[idst.at[j0 - 2 + i]],
                                      ssems[i]).wait()
            pltpu.async_copy(gsh.at[isrc.at[j0 + i]], gbufs[i], gsems[i])
        for i in range(2):
            pltpu.make_async_copy(gsh.at[isrc.at[j0 + i]], gbufs[i],
                                  gsems[i]).wait()
            pltpu.async_copy(gbufs[i], acc.at[idst.at[j0 + i]], ssems[i],
                             add=True)

    for i in range(2):
        pltpu.make_async_copy(gbufs[i], acc.at[idst.at[NCH2 - 2 + i]],
                              ssems[i]).wait()

    plsc.subcore_barrier()
    pltpu.sync_copy(acc.at[pl.ds(s * WROWS, WROWS)],
                    s_out.at[c, pl.ds(s * WROWS, WROWS)])

    @pl.when(s == 0)
    def _():
        pltpu.sync_copy(acc.at[pl.ds(NS * WROWS, N - NS * WROWS)],
                        s_out.at[c, pl.ds(NS * WROWS, N - NS * WROWS)])


def _sc_deg(src4, ones_16, zeros_16):
    return pl.kernel(
        _sc_deg_body,
        mesh=_mesh(),
        out_type=jax.ShapeDtypeStruct((NC, N, 16), jnp.float32),
        scratch_types=[
            pltpu.VMEM((NCH, CH), jnp.int32),
            pltpu.VMEM((CH, 16), jnp.float32),
            pltpu.VMEM_SHARED((ACC, 16), jnp.float32),
            pltpu.SemaphoreType.DMA,
        ],
        compiler_params=_SC_PARAMS,
    )(src4, ones_16, zeros_16)


def _sc_deg_body(src_hbm, ones_hbm, zeros_hbm, d_out, isrc, ones_v, acc, sem):
    c = lax.axis_index("c")
    s = lax.axis_index("s")
    pltpu.sync_copy(zeros_hbm.at[pl.ds(s * ZROWS, ZROWS)],
                    acc.at[pl.ds(s * ZROWS, ZROWS)])
    pltpu.sync_copy(src_hbm.at[c, s], isrc)
    pltpu.sync_copy(ones_hbm, ones_v)
    plsc.subcore_barrier()

    @pl.loop(0, NCH)
    def _(j):
        pltpu.sync_copy(ones_v, acc.at[isrc.at[j]], add=True)

    plsc.subcore_barrier()
    pltpu.sync_copy(acc.at[pl.ds(s * WROWS, WROWS)],
                    d_out.at[c, pl.ds(s * WROWS, WROWS)])

    @pl.when(s == 0)
    def _():
        pltpu.sync_copy(acc.at[pl.ds(NS * WROWS, N - NS * WROWS)],
                        d_out.at[c, pl.ds(NS * WROWS, N - NS * WROWS)])


BM = 2000
BD = 2000


def _mm_body(x_ref, w_ref, o_ref):
    o_ref[0] = jnp.dot(x_ref[...], w_ref[0],
                       preferred_element_type=jnp.float32)


def _matmul(x, W):
    return pl.pallas_call(
        _mm_body,
        grid=(N // BM, K),
        in_specs=[
            pl.BlockSpec((BM, D_IN), lambda i, k: (i, 0)),
            pl.BlockSpec((1, D_IN, H), lambda i, k: (k, 0, 0)),
        ],
        out_specs=pl.BlockSpec((1, BM, H), lambda i, k: (k, i, 0)),
        out_shape=jax.ShapeDtypeStruct((K, N, H), jnp.float32),
    )(x, W)


def _pre_body(deg_ref, a7_ref, dinv_ref, g_ref):
    deg = deg_ref[0, :, 0:1] + deg_ref[1, :, 0:1]
    dinv = jnp.where(deg > 0, lax.rsqrt(jnp.maximum(deg, 1.0)), 0.0)
    dinv_ref[...] = dinv
    g = dinv * a7_ref[0]
    g_ref[0] = g[:, :HH]
    g_ref[1] = g[:, HH:]


def _pre(deg_parts, a):
    return pl.pallas_call(
        _pre_body,
        grid=(N // BD,),
        in_specs=[
            pl.BlockSpec((NC, BD, 16), lambda i: (0, i, 0)),
            pl.BlockSpec((1, BD, H), lambda i: (K - 1, i, 0)),
        ],
        out_specs=[
            pl.BlockSpec((BD, 1), lambda i: (i, 0)),
            pl.BlockSpec((NC, BD, HH), lambda i: (0, i, 0)),
        ],
        out_shape=[
            jax.ShapeDtypeStruct((N, 1), jnp.float32),
            jax.ShapeDtypeStruct((NC, N, HH), jnp.float32),
        ],
    )(deg_parts, a)


def _dense_body(a_ref, s_ref, dinv_ref, bk2_ref, bk_ref, g_ref):
    ssum = jnp.concatenate([s_ref[0], s_ref[1]], axis=-1)
    dinv = dinv_ref[...]
    bk = a_ref[0] - 2.0 * dinv * ssum - bk2_ref[...]
    bk_ref[...] = bk
    g = dinv * bk
    g_ref[0] = g[:, :HH]
    g_ref[1] = g[:, HH:]


def _dense_body_nob(a_ref, s_ref, dinv_ref, bk_ref, g_ref):
    ssum = jnp.concatenate([s_ref[0], s_ref[1]], axis=-1)
    dinv = dinv_ref[...]
    bk = a_ref[0] - 2.0 * dinv * ssum
    bk_ref[...] = bk
    g = dinv * bk
    g_ref[0] = g[:, :HH]
    g_ref[1] = g[:, HH:]


def _dense(k, a, s_parts, dinv, bk2):
    in_specs = [
        pl.BlockSpec((1, BD, H), lambda i, k=k: (k, i, 0)),
        pl.BlockSpec((NC, BD, HH), lambda i: (0, i, 0)),
        pl.BlockSpec((BD, 1), lambda i: (i, 0)),
    ]
    args = [a, s_parts, dinv]
    if bk2 is None:
        body = _dense_body_nob
    else:
        body = _dense_body
        in_specs.append(pl.BlockSpec((BD, H), lambda i: (i, 0)))
        args.append(bk2)
    return pl.pallas_call(
        body,
        grid=(N // BD,),
        in_specs=in_specs,
        out_specs=[
            pl.BlockSpec((BD, H), lambda i: (i, 0)),
            pl.BlockSpec((NC, BD, HH), lambda i: (0, i, 0)),
        ],
        out_shape=[
            jax.ShapeDtypeStruct((N, H), jnp.float32),
            jax.ShapeDtypeStruct((NC, N, HH), jnp.float32),
        ],
    )(*args)


def _final_body(a_ref, s_ref, dinv_ref, b2_ref, bias_ref, o_ref):
    ssum = jnp.concatenate([s_ref[0], s_ref[1]], axis=-1)
    o_ref[...] = jnp.maximum(
        a_ref[0] - dinv_ref[...] * ssum - b2_ref[...] + bias_ref[...], 0.0)


def _final(a, s_parts, dinv, b2, bias2d):
    return pl.pallas_call(
        _final_body,
        grid=(N // BD,),
        in_specs=[
            pl.BlockSpec((1, BD, H), lambda i: (0, i, 0)),
            pl.BlockSpec((NC, BD, HH), lambda i: (0, i, 0)),
            pl.BlockSpec((BD, 1), lambda i: (i, 0)),
            pl.BlockSpec((BD, H), lambda i: (i, 0)),
            pl.BlockSpec((1, H), lambda i: (0, 0)),
        ],
        out_specs=pl.BlockSpec((BD, H), lambda i: (i, 0)),
        out_shape=jax.ShapeDtypeStruct((N, H), jnp.float32),
    )(a, s_parts, dinv, b2, bias2d)


def kernel(x, edge_index, W, b):
    src = edge_index[0].astype(jnp.int32)
    dst = edge_index[1].astype(jnp.int32)
    pad = EP - E
    shape4 = (NC, NS, NCH, CH)
    src_s = jnp.concatenate([src, jnp.full((pad,), TRASH, jnp.int32)]).reshape(shape4)
    pad2 = EP2 - E
    shape3 = (NS, NCH2, CH)
    src_g = jnp.concatenate([src, jnp.zeros((pad2,), jnp.int32)]).reshape(shape3)
    dst_s = jnp.concatenate([dst, jnp.full((pad2,), TRASH, jnp.int32)]).reshape(shape3)

    bias2d = b.reshape(1, H)
    zeros_h = jnp.zeros((ACC, HH), jnp.float32)
    zeros_16 = jnp.zeros((ACC, 16), jnp.float32)
    ones_16 = jnp.ones((CH, 16), jnp.float32)

    deg_parts = _sc_deg(src_s, ones_16, zeros_16)
    a = _matmul(x, W)
    dinv, g = _pre(deg_parts, a)

    b_prev2 = None
    b_prev1 = a[K - 1]
    for k in range(K - 2, 0, -1):
        s_parts = _sc_prop(g, src_g, dst_s, zeros_h)
        bk, g = _dense(k, a, s_parts, dinv, b_prev2)
        b_prev2, b_prev1 = b_prev1, bk

    s_parts = _sc_prop(g, src_g, dst_s, zeros_h)
    return _final(a, s_parts, dinv, b_prev2, bias2d)
